# speedup vs baseline: 1.8393x; 1.8393x over previous
import functools
import math

import jax
import jax.numpy as jnp
from jax import lax
from jax.experimental import pallas as pl
from jax.experimental.pallas import tpu as pltpu
from jax.experimental.pallas import tpu_sc as plsc

VOCAB = 100000
D = 512
BATCH = 4096
SEQ = 200
N = BATCH * SEQ
NC, NS = 2, 16
NW = NC * NS
ROWS_W = N // NW
CHUNK = 128
NCHUNK = ROWS_W // CHUNK
SCALE = math.sqrt(float(D))

_ROWS_BLK = 2000


def _scale_body(t_ref, o_ref):
    o_ref[...] = t_ref[...] * jnp.float32(SCALE)


def _scale_table(table):
    return pl.pallas_call(
        _scale_body,
        grid=(VOCAB // _ROWS_BLK,),
        in_specs=[pl.BlockSpec((_ROWS_BLK, D), lambda i: (i, 0))],
        out_specs=pl.BlockSpec((_ROWS_BLK, D), lambda i: (i, 0)),
        out_shape=jax.ShapeDtypeStruct((VOCAB, D), jnp.float32),
    )(table)


_mesh = plsc.VectorSubcoreMesh(
    core_axis_name="c", subcore_axis_name="s", num_cores=NC, num_subcores=NS
)


@functools.partial(
    pl.kernel,
    out_type=jax.ShapeDtypeStruct((N, D), jnp.float32),
    mesh=_mesh,
    scratch_types=[
        pltpu.VMEM((CHUNK,), jnp.int32),
        pltpu.VMEM((CHUNK, D), jnp.float32),
        pltpu.SemaphoreType.DMA,
    ],
)
def _sc_gather(table_hbm, idx_hbm, out_hbm, idx_v, rows_v, sem):
    wid = lax.axis_index("s") * NC + lax.axis_index("c")
    base = wid * ROWS_W

    @pl.loop(0, NCHUNK)
    def _chunk(c):
        r0 = base + c * CHUNK
        pltpu.sync_copy(idx_hbm.at[pl.ds(r0, CHUNK)], idx_v)
        pltpu.async_copy(table_hbm.at[idx_v], rows_v, sem).wait()
        pltpu.sync_copy(rows_v, out_hbm.at[pl.ds(r0, CHUNK)])


def kernel(x, table):
    scaled = _scale_table(table)
    out = _sc_gather(scaled, x.reshape(N))
    return out.reshape(BATCH, SEQ, D)


# speedup vs baseline: 2.1112x; 1.1478x over previous
import functools
import math

import jax
import jax.numpy as jnp
from jax import lax
from jax.experimental import pallas as pl
from jax.experimental.pallas import tpu as pltpu
from jax.experimental.pallas import tpu_sc as plsc

VOCAB = 100000
D = 512
BATCH = 4096
SEQ = 200
N = BATCH * SEQ
NC, NS = 2, 16
NW = NC * NS
ROWS_W = N // NW
CHUNK = 80
NCHUNK = ROWS_W // CHUNK
SCALE = math.sqrt(float(D))

_ROWS_BLK = 2000


def _scale_body(t_ref, o_ref):
    o_ref[...] = t_ref[...] * jnp.float32(SCALE)


def _scale_table(table):
    return pl.pallas_call(
        _scale_body,
        grid=(VOCAB // _ROWS_BLK,),
        in_specs=[pl.BlockSpec((_ROWS_BLK, D), lambda i: (i, 0))],
        out_specs=pl.BlockSpec((_ROWS_BLK, D), lambda i: (i, 0)),
        out_shape=jax.ShapeDtypeStruct((VOCAB, D), jnp.float32),
    )(table)


_mesh = plsc.VectorSubcoreMesh(
    core_axis_name="c", subcore_axis_name="s", num_cores=NC, num_subcores=NS
)


@functools.partial(
    pl.kernel,
    out_type=jax.ShapeDtypeStruct((N, D), jnp.float32),
    mesh=_mesh,
    scratch_types=[
        pltpu.VMEM((ROWS_W,), jnp.int32),
        pltpu.VMEM((CHUNK, D), jnp.float32),
        pltpu.VMEM((CHUNK, D), jnp.float32),
        pltpu.SemaphoreType.DMA,
        pltpu.SemaphoreType.DMA,
        pltpu.SemaphoreType.DMA,
        pltpu.SemaphoreType.DMA,
    ],
)
def _sc_gather(table_hbm, idx_hbm, out_hbm, idx_v, rows0, rows1, g0, g1, s0, s1):
    wid = lax.axis_index("s") * NC + lax.axis_index("c")
    base = wid * ROWS_W

    pltpu.sync_copy(idx_hbm.at[pl.ds(base, ROWS_W)], idx_v)

    def idx_slice(c):
        return idx_v.at[pl.ds(c * CHUNK, CHUNK)]

    def out_slice(c):
        return out_hbm.at[pl.ds(base + c * CHUNK, CHUNK)]

    pltpu.async_copy(table_hbm.at[idx_slice(0)], rows0, g0)

    @pl.loop(0, NCHUNK, step=2)
    def _pair(c):
        @pl.when(c > 0)
        def _():
            pltpu.make_async_copy(rows1, out_slice(c - 1), s1).wait()

        pltpu.async_copy(table_hbm.at[idx_slice(c + 1)], rows1, g1)
        pltpu.make_async_copy(table_hbm.at[idx_slice(c)], rows0, g0).wait()
        pltpu.async_copy(rows0, out_slice(c), s0)

        @pl.when(c + 2 < NCHUNK)
        def _():
            pltpu.make_async_copy(rows0, out_slice(c), s0).wait()
            pltpu.async_copy(table_hbm.at[idx_slice(c + 2)], rows0, g0)

        pltpu.make_async_copy(table_hbm.at[idx_slice(c + 1)], rows1, g1).wait()
        pltpu.async_copy(rows1, out_slice(c + 1), s1)

    pltpu.make_async_copy(rows0, out_slice(NCHUNK - 2), s0).wait()
    pltpu.make_async_copy(rows1, out_slice(NCHUNK - 1), s1).wait()


def kernel(x, table):
    scaled = _scale_table(table)
    out = _sc_gather(scaled, x.reshape(N))
    return out.reshape(BATCH, SEQ, D)
